# baseline (speedup 1.0000x reference)
import functools

import jax
import jax.numpy as jnp
from jax import lax
from jax.experimental import pallas as pl
from jax.experimental.pallas import tpu as pltpu
from jax.experimental.pallas import tpu_sc as plsc

_NBUF = 2


@functools.lru_cache(maxsize=None)
def _make_gather(V, D, DP, B):
    info = plsc.get_sparse_core_info()
    NC, NS = info.num_cores, info.num_subcores
    NW = NC * NS
    assert B % NW == 0
    b_per_w = B // NW
    C = 256
    assert b_per_w % (C * _NBUF) == 0
    n_chunks = b_per_w // C
    n_groups = n_chunks // _NBUF
    mesh = plsc.VectorSubcoreMesh(core_axis_name="c", subcore_axis_name="s")

    @functools.partial(
        pl.kernel,
        mesh=mesh,
        out_type=jax.ShapeDtypeStruct((B, DP), jnp.float32),
        compiler_params=pltpu.CompilerParams(use_tc_tiling_on_sc=True),
        scratch_types=[
            [pltpu.VMEM((C,), jnp.int32)] * _NBUF,
            [pltpu.VMEM((C, DP), jnp.float32)] * _NBUF,
            [pltpu.SemaphoreType.DMA] * _NBUF,
            [pltpu.SemaphoreType.DMA] * _NBUF,
        ],
    )
    def k(idx_hbm, table_hbm, out_hbm, idx_v, rows_v, gsem, ssem):
        wid = lax.axis_index("s") * NC + lax.axis_index("c")
        w_base = wid * b_per_w

        def start_gather(chunk, b):
            pltpu.sync_copy(idx_hbm.at[pl.ds(w_base + chunk * C, C)],
                            idx_v[b])
            pltpu.async_copy(table_hbm.at[idx_v[b]], rows_v[b], gsem[b])

        def wait_gather(b):
            pltpu.make_async_copy(table_hbm.at[idx_v[b]], rows_v[b],
                                  gsem[b]).wait()

        def start_store(chunk, b):
            pltpu.async_copy(rows_v[b],
                             out_hbm.at[pl.ds(w_base + chunk * C, C)], ssem[b])

        def wait_store(chunk, b):
            pltpu.make_async_copy(rows_v[b],
                                  out_hbm.at[pl.ds(w_base + chunk * C, C)],
                                  ssem[b]).wait()

        for b in range(_NBUF):
            start_gather(b, b)

        def body(g, carry):
            for b in range(_NBUF):
                i = g * _NBUF + b
                wait_gather(b)
                start_store(i, b)
                wait_store(i, b)
                start_gather(i + _NBUF, b)
            return carry

        lax.fori_loop(0, n_groups - 1, body, 0)

        for b in range(_NBUF):
            i = (n_groups - 1) * _NBUF + b
            wait_gather(b)
            pltpu.sync_copy(rows_v[b], out_hbm.at[pl.ds(w_base + i * C, C)])

    return k


def kernel(input_x, table):
    Bt, H = input_x.shape
    V, D = table.shape
    DP = 128
    tpad = jnp.pad(table, ((0, 0), (0, DP - D)))
    idx = input_x.reshape(-1)
    out = _make_gather(V, D, DP, idx.shape[0])(idx, tpad)
    return out[:, :D].reshape(Bt, H, D)
